# Initial kernel scaffold; baseline (speedup 1.0000x reference)
import jax
import jax.numpy as jnp
from jax.experimental import pallas as pl


def kernel(token_states, set_states, desc_router, token_to_sets, W_q, b_q):
    raise NotImplementedError("write your pallas kernel here")


# speedup vs baseline: 5.1902x; 5.1902x over previous
import functools
import numpy as np
import jax
import jax.numpy as jnp
from jax.experimental import pallas as pl
from jax.experimental.pallas import tpu as pltpu

D_MODEL = 1024
NUM_SETS = 64
K_TOP = 8
NEG = -1e30


def _router_body(x_ref, set_ref, desc_ref, tts_ref, wq_ref, bq_ref,
                 out_ref):
    desc = desc_ref[0]
    scale = 1.0 / np.sqrt(D_MODEL)

    x = x_ref[0]
    q = jax.lax.dot_general(
        x, wq_ref[...], (((1,), (1,)), ((), ())),
        preferred_element_type=jnp.float32) + bq_ref[...]
    scores = jax.lax.dot_general(
        q, desc, (((1,), (1,)), ((), ())),
        preferred_element_type=jnp.float32) * scale

    tts = tts_ref[...]
    col_ids = jax.lax.broadcasted_iota(jnp.int32, scores.shape, 1)
    mask = jnp.zeros(scores.shape, dtype=jnp.bool_)
    for j in range(16):
        mask = mask | (tts[:, j:j + 1] == col_ids)
    masked = jnp.where(mask, scores, NEG)

    cur = masked
    thr = None
    for _ in range(K_TOP):
        thr = jnp.max(cur, axis=-1, keepdims=True)
        cur = jnp.where(cur >= thr, NEG, cur)
    keep = jnp.where(masked >= thr, masked, NEG)

    m = jnp.max(keep, axis=-1, keepdims=True)
    e = jnp.exp(keep - m)
    w = e / jnp.sum(e, axis=-1, keepdims=True)

    out_ref[0] = jax.lax.dot_general(
        w, set_ref[0], (((1,), (0,)), ((), ())),
        preferred_element_type=jnp.float32)


@functools.partial(jax.jit, static_argnames=("interpret",))
def _run(token_states, set_states, desc_router, token_to_sets, W_q, b_q2,
         interpret=False):
    batch, seq_len, d = token_states.shape
    blk = 512
    nb = seq_len // blk
    grid = (batch, nb)
    return pl.pallas_call(
        _router_body,
        grid=grid,
        in_specs=[
            pl.BlockSpec((1, blk, d), lambda b, i: (b, i, 0)),
            pl.BlockSpec((1, NUM_SETS, d), lambda b, i: (b, 0, 0)),
            pl.BlockSpec((1, NUM_SETS, d), lambda b, i: (b, 0, 0)),
            pl.BlockSpec((blk, 16), lambda b, i: (i, 0)),
            pl.BlockSpec((d, d), lambda b, i: (0, 0)),
            pl.BlockSpec((1, d), lambda b, i: (0, 0)),
        ],
        out_specs=pl.BlockSpec((1, blk, d), lambda b, i: (b, i, 0)),
        out_shape=jax.ShapeDtypeStruct((batch, seq_len, d), jnp.float32),
        interpret=interpret,
    )(token_states, set_states, desc_router, token_to_sets, W_q, b_q2)


def kernel(token_states, set_states, desc_router, token_to_sets, W_q, b_q):
    return _run(token_states, set_states, desc_router,
                token_to_sets.astype(jnp.int32), W_q,
                b_q.reshape(1, -1))
